# Initial kernel scaffold; baseline (speedup 1.0000x reference)
import jax
import jax.numpy as jnp
from jax.experimental import pallas as pl


def kernel(bias_table, relative_position_index):
    raise NotImplementedError("write your pallas kernel here")


# speedup vs baseline: 4.9654x; 4.9654x over previous
import functools

import jax
import jax.numpy as jnp
from jax import lax
from jax.experimental import pallas as pl
from jax.experimental.pallas import tpu as pltpu
from jax.experimental.pallas import tpu_sc as plsc

_NUM_CORES = 2
_NUM_SUBCORES = 16
_NUM_WORKERS = _NUM_CORES * _NUM_SUBCORES
_LANES = 16


def _make_sc_gather(num_offsets: int, num_heads: int, num_pos: int):
    assert num_pos % (_NUM_WORKERS * _LANES) == 0
    chunk = num_pos // _NUM_WORKERS
    table_words = num_offsets * num_heads
    mesh = plsc.VectorSubcoreMesh(
        core_axis_name="c", subcore_axis_name="s",
        num_cores=_NUM_CORES, num_subcores=_NUM_SUBCORES)

    @functools.partial(
        pl.kernel,
        out_type=jax.ShapeDtypeStruct((num_heads, num_pos), jnp.float32),
        mesh=mesh,
        scratch_types=[
            pltpu.VMEM((table_words,), jnp.float32),
            pltpu.VMEM((chunk,), jnp.int32),
            pltpu.VMEM((num_heads, chunk), jnp.float32),
        ],
        compiler_params=pltpu.CompilerParams(needs_layout_passes=False),
    )
    def sc_gather(table_hbm, idx_hbm, out_hbm, table_v, idx_v, out_v):
        wid = lax.axis_index("s") * _NUM_CORES + lax.axis_index("c")
        base = wid * chunk
        pltpu.sync_copy(table_hbm, table_v)
        pltpu.sync_copy(idx_hbm.at[pl.ds(base, chunk)], idx_v)

        def body(g, carry):
            iv = idx_v[pl.ds(g * _LANES, _LANES)]
            flat_base = iv * num_heads
            for h in range(num_heads):
                out_v[h, pl.ds(g * _LANES, _LANES)] = plsc.load_gather(
                    table_v, [flat_base + h])
            return carry

        lax.fori_loop(0, chunk // _LANES, body, 0)

        for h in range(num_heads):
            pltpu.sync_copy(out_v.at[h], out_hbm.at[h, pl.ds(base, chunk)])

    return sc_gather


def kernel(bias_table, relative_position_index):
    num_offsets, num_heads = bias_table.shape
    w2 = relative_position_index.shape[0]
    num_pos = w2 * relative_position_index.shape[1]
    table_flat = bias_table.reshape(-1).astype(jnp.float32)
    idx_flat = relative_position_index.reshape(-1).astype(jnp.int32)
    out = _make_sc_gather(num_offsets, num_heads, num_pos)(table_flat, idx_flat)
    return out.reshape(num_heads, w2, relative_position_index.shape[1])
